# baseline (speedup 1.0000x reference)
import functools
import jax
import jax.numpy as jnp
from jax import lax
from jax.experimental import pallas as pl
from jax.experimental.pallas import tpu as pltpu
from jax.experimental.pallas import tpu_sc as plsc

_N = 10000
_E = 320000
_D = 128
_NC = 2
_NS = 16
_NW = _NC * _NS
_GRP = 125
_CH = 16
_NCH = -(-_E // (_NW * _GRP * _CH))
_G = _NCH * _CH
_E_PAD = _NW * _G * _GRP
_N_ACC = 10112
_RPT = _N_ACC // _NS


def _edge_pass(table, src_idx, dst_idx):
    mesh = plsc.VectorSubcoreMesh(core_axis_name="c", subcore_axis_name="s")

    @functools.partial(
        pl.kernel,
        mesh=mesh,
        out_type=jax.ShapeDtypeStruct((_NC, _N_ACC, _D), jnp.float32),
        scratch_types=[
            pltpu.VMEM((_CH, _GRP), jnp.int32),
            pltpu.VMEM((_CH, _GRP), jnp.int32),
            pltpu.VMEM((_CH, _GRP), jnp.int32),
            pltpu.VMEM((_CH, _GRP), jnp.int32),
            pltpu.VMEM((_GRP, _D), jnp.float32),
            pltpu.VMEM((_GRP, _D), jnp.float32),
            pltpu.VMEM_SHARED((_N_ACC, _D), jnp.float32),
            pltpu.SemaphoreType.DMA,
            pltpu.SemaphoreType.DMA,
            pltpu.SemaphoreType.DMA,
            pltpu.SemaphoreType.DMA,
        ],
    )
    def k(table_hbm, src_hbm, dst_hbm, out_hbm, si0, si1, di0, di1,
          rows0, rows1, acc, semr0, semr1, semi0, semi1):
        c = lax.axis_index("c")
        s = lax.axis_index("s")
        wid = c * _NS + s
        base = s * _RPT
        sbuf, dbuf, semi = (si0, si1), (di0, di1), (semi0, semi1)
        rbuf, semr = (rows0, rows1), (semr0, semr1)
        zeros16 = jnp.zeros((16,), jnp.float32)

        def idx_start(ch):
            pltpu.async_copy(src_hbm.at[wid, ch], sbuf[ch % 2], semi[ch % 2])
            pltpu.async_copy(dst_hbm.at[wid, ch], dbuf[ch % 2], semi[ch % 2])

        def idx_wait(ch):
            pltpu.make_async_copy(
                src_hbm.at[wid, ch], sbuf[ch % 2], semi[ch % 2]).wait()
            pltpu.make_async_copy(
                dst_hbm.at[wid, ch], dbuf[ch % 2], semi[ch % 2]).wait()

        idx_start(0)

        def zrow(i, carry):
            for j in range(_D // 16):
                rows0[i, pl.ds(j * 16, 16)] = zeros16
            return carry

        lax.fori_loop(0, _GRP, zrow, 0)
        for bb in range(_RPT // 120):
            pltpu.sync_copy(rows0.at[pl.ds(0, 120)],
                            acc.at[pl.ds(base + bb * 120, 120)])
        rem = _RPT % 120
        if rem:
            pltpu.sync_copy(rows0.at[pl.ds(0, rem)],
                            acc.at[pl.ds(base + _RPT - rem, rem)])
        plsc.subcore_barrier()

        idx_wait(0)
        pltpu.async_copy(table_hbm.at[di0.at[0]], rows0, semr0)

        for ch in range(_NCH):
            sidx, didx = sbuf[ch % 2], dbuf[ch % 2]
            if ch + 1 < _NCH:
                idx_start(ch + 1)
            for j in range(_CH):
                g = ch * _CH + j
                b = g % 2
                if g + 1 < _G:
                    if j + 1 < _CH:
                        dref = didx.at[j + 1]
                    else:
                        idx_wait(ch + 1)
                        dref = dbuf[(ch + 1) % 2].at[0]
                    pltpu.async_copy(table_hbm.at[dref],
                                     rbuf[1 - b], semr[1 - b])
                pltpu.make_async_copy(table_hbm.at[didx.at[j]],
                                      rbuf[b], semr[b]).wait()
                pltpu.sync_copy(rbuf[b], acc.at[sidx.at[j]], add=True)

        plsc.subcore_barrier()
        pltpu.sync_copy(acc.at[pl.ds(base, _RPT)],
                        out_hbm.at[c, pl.ds(base, _RPT)])

    return k(table, src_idx, dst_idx)


def _xwt(x, w):
    return lax.dot_general(x, w, (((1,), (1,)), ((), ())),
                           preferred_element_type=jnp.float32)


def _mm_relu_k(x_ref, w_ref, b_ref, o_ref):
    o_ref[...] = jnp.maximum(_xwt(x_ref[...], w_ref[...]) + b_ref[...], 0.0)


def _merge_mm_k(a_ref, w_ref, b_ref, o_ref):
    o_ref[...] = _xwt(a_ref[0] + a_ref[1], w_ref[...]) + b_ref[...]


def _linear_relu(x, w, b):
    n = x.shape[0]
    blk = 1000
    return pl.pallas_call(
        _mm_relu_k,
        grid=(n // blk,),
        in_specs=[
            pl.BlockSpec((blk, _D), lambda i: (i, 0)),
            pl.BlockSpec((_D, _D), lambda i: (0, 0)),
            pl.BlockSpec((1, _D), lambda i: (0, 0)),
        ],
        out_specs=pl.BlockSpec((blk, _D), lambda i: (i, 0)),
        out_shape=jax.ShapeDtypeStruct((n, _D), jnp.float32),
    )(x, w, b)


def _merge_linear(acc, w, b):
    k = w.shape[0]
    blk = 1000
    return pl.pallas_call(
        _merge_mm_k,
        grid=(_N // blk,),
        in_specs=[
            pl.BlockSpec((_NC, blk, _D), lambda i: (0, i, 0)),
            pl.BlockSpec((k, _D), lambda i: (0, 0)),
            pl.BlockSpec((1, k), lambda i: (0, 0)),
        ],
        out_specs=pl.BlockSpec((blk, k), lambda i: (i, 0)),
        out_shape=jax.ShapeDtypeStruct((_N, k), jnp.float32),
    )(acc, w, b)


def kernel(node_features, edge_index, rank_mapping, W1, b1, W2, b2, Wf, bf):
    del rank_mapping
    f = node_features[0].astype(jnp.float32)
    src_p = edge_index[0, 0, :].astype(jnp.int32).reshape(
        _NW, _NCH, _CH, _GRP)
    dst_p = edge_index[0, 1, :].astype(jnp.int32).reshape(
        _NW, _NCH, _CH, _GRP)

    h1 = _linear_relu(f, W1, b1.reshape(1, _D))
    acc1 = _edge_pass(h1, src_p, dst_p)
    h2 = _merge_linear(acc1, W2, b2.reshape(1, _D))
    acc2 = _edge_pass(h2, src_p, dst_p)
    out = _merge_linear(acc2, Wf, bf.reshape(1, -1))
    return out[None]


# speedup vs baseline: 1.0433x; 1.0433x over previous
import functools
import jax
import jax.numpy as jnp
from jax import lax
from jax.experimental import pallas as pl
from jax.experimental.pallas import tpu as pltpu
from jax.experimental.pallas import tpu_sc as plsc

_N = 10000
_E = 320000
_D = 128
_NC = 2
_NS = 16
_NW = _NC * _NS
_GRP = 125
_CH = 16
_NCH = -(-_E // (_NW * _GRP * _CH))
_G = _NCH * _CH
_E_PAD = _NW * _G * _GRP
_N_ACC = 10112
_RPT = _N_ACC // _NS


def _edge_pass(table, eidx):
    mesh = plsc.VectorSubcoreMesh(core_axis_name="c", subcore_axis_name="s")

    @functools.partial(
        pl.kernel,
        mesh=mesh,
        out_type=jax.ShapeDtypeStruct((_NC, _N_ACC, _D), jnp.float32),
        scratch_types=[
            pltpu.VMEM((_CH, _GRP), jnp.int32),
            pltpu.VMEM((_CH, _GRP), jnp.int32),
            pltpu.VMEM((_CH, _GRP), jnp.int32),
            pltpu.VMEM((_CH, _GRP), jnp.int32),
            pltpu.VMEM((_GRP, _D), jnp.float32),
            pltpu.VMEM((_GRP, _D), jnp.float32),
            pltpu.VMEM_SHARED((_N_ACC, _D), jnp.float32),
            pltpu.SemaphoreType.DMA,
            pltpu.SemaphoreType.DMA,
            pltpu.SemaphoreType.DMA,
            pltpu.SemaphoreType.DMA,
        ],
    )
    def k(table_hbm, eidx_hbm, out_hbm, si0, si1, di0, di1,
          rows0, rows1, acc, semr0, semr1, semi0, semi1):
        c = lax.axis_index("c")
        s = lax.axis_index("s")
        wid = c * _NS + s
        base = s * _RPT
        sbuf, dbuf, semi = (si0, si1), (di0, di1), (semi0, semi1)
        rbuf, semr = (rows0, rows1), (semr0, semr1)
        zeros16 = jnp.zeros((16,), jnp.float32)

        def idx_start(ch):
            pltpu.async_copy(
                eidx_hbm.at[0, wid, ch], sbuf[ch % 2], semi[ch % 2])
            pltpu.async_copy(
                eidx_hbm.at[1, wid, ch], dbuf[ch % 2], semi[ch % 2])

        def idx_wait(ch):
            pltpu.make_async_copy(
                eidx_hbm.at[0, wid, ch], sbuf[ch % 2], semi[ch % 2]).wait()
            pltpu.make_async_copy(
                eidx_hbm.at[1, wid, ch], dbuf[ch % 2], semi[ch % 2]).wait()

        idx_start(0)

        def zrow(i, carry):
            for j in range(_D // 16):
                rows0[i, pl.ds(j * 16, 16)] = zeros16
            return carry

        lax.fori_loop(0, _GRP, zrow, 0)
        for bb in range(_RPT // 120):
            pltpu.sync_copy(rows0.at[pl.ds(0, 120)],
                            acc.at[pl.ds(base + bb * 120, 120)])
        rem = _RPT % 120
        if rem:
            pltpu.sync_copy(rows0.at[pl.ds(0, rem)],
                            acc.at[pl.ds(base + _RPT - rem, rem)])
        plsc.subcore_barrier()

        idx_wait(0)
        pltpu.async_copy(table_hbm.at[di0.at[0]], rows0, semr0)

        for ch in range(_NCH):
            sidx, didx = sbuf[ch % 2], dbuf[ch % 2]
            if ch + 1 < _NCH:
                idx_start(ch + 1)
            for j in range(_CH):
                g = ch * _CH + j
                b = g % 2
                if g + 1 < _G:
                    if j + 1 < _CH:
                        dref = didx.at[j + 1]
                    else:
                        idx_wait(ch + 1)
                        dref = dbuf[(ch + 1) % 2].at[0]
                    pltpu.async_copy(table_hbm.at[dref],
                                     rbuf[1 - b], semr[1 - b])
                pltpu.make_async_copy(table_hbm.at[didx.at[j]],
                                      rbuf[b], semr[b]).wait()
                pltpu.sync_copy(rbuf[b], acc.at[sidx.at[j]], add=True)

        plsc.subcore_barrier()
        pltpu.sync_copy(acc.at[pl.ds(base, _RPT)],
                        out_hbm.at[c, pl.ds(base, _RPT)])

    return k(table, eidx)


def _xwt(x, w):
    return lax.dot_general(x, w, (((1,), (1,)), ((), ())),
                           preferred_element_type=jnp.float32)


def _mm_relu_k(x_ref, w_ref, b_ref, o_ref):
    o_ref[...] = jnp.maximum(_xwt(x_ref[...], w_ref[...]) + b_ref[...], 0.0)


def _merge_mm_k(a_ref, w_ref, b_ref, o_ref):
    o_ref[...] = _xwt(a_ref[0] + a_ref[1], w_ref[...]) + b_ref[...]


def _linear_relu(x, w, b):
    n = x.shape[0]
    blk = 1000
    return pl.pallas_call(
        _mm_relu_k,
        grid=(n // blk,),
        in_specs=[
            pl.BlockSpec((blk, _D), lambda i: (i, 0)),
            pl.BlockSpec((_D, _D), lambda i: (0, 0)),
            pl.BlockSpec((1, _D), lambda i: (0, 0)),
        ],
        out_specs=pl.BlockSpec((blk, _D), lambda i: (i, 0)),
        out_shape=jax.ShapeDtypeStruct((n, _D), jnp.float32),
    )(x, w, b)


def _merge_linear(acc, w, b):
    k = w.shape[0]
    blk = 1000
    return pl.pallas_call(
        _merge_mm_k,
        grid=(_N // blk,),
        in_specs=[
            pl.BlockSpec((_NC, blk, _D), lambda i: (0, i, 0)),
            pl.BlockSpec((k, _D), lambda i: (0, 0)),
            pl.BlockSpec((1, k), lambda i: (0, 0)),
        ],
        out_specs=pl.BlockSpec((blk, k), lambda i: (i, 0)),
        out_shape=jax.ShapeDtypeStruct((_N, k), jnp.float32),
    )(acc, w, b)


def kernel(node_features, edge_index, rank_mapping, W1, b1, W2, b2, Wf, bf):
    del rank_mapping
    f = node_features[0].astype(jnp.float32)
    e_p = edge_index[0].astype(jnp.int32).reshape(2, _NW, _NCH, _CH, _GRP)

    h1 = _linear_relu(f, W1, b1.reshape(1, _D))
    acc1 = _edge_pass(h1, e_p)
    h2 = _merge_linear(acc1, W2, b2.reshape(1, _D))
    acc2 = _edge_pass(h2, e_p)
    out = _merge_linear(acc2, Wf, bf.reshape(1, -1))
    return out[None]
